# Initial kernel scaffold; baseline (speedup 1.0000x reference)
import jax
import jax.numpy as jnp
from jax.experimental import pallas as pl


def kernel(edge_index, edge_weight, emb_users, emb_items, W1, b1, W2, b2):
    raise NotImplementedError("write your pallas kernel here")


# speedup vs baseline: 9.0717x; 9.0717x over previous
import functools

import jax
import jax.numpy as jnp
from jax import lax
from jax.experimental import pallas as pl
from jax.experimental.pallas import tpu as pltpu
from jax.experimental.pallas import tpu_sc as plsc

N_NODES = 50000
DIM = 64
HALF = 25000
NC, NS, LANES = 2, 16, 16
CHUNK = 128
E_EDGES = 800000
CHUNKS = -(-E_EDGES // CHUNK)
CPT = -(-CHUNKS // NS)
E_PAD = NS * CPT * CHUNK
ACC_ROWS = 25088
ROWS_PT = ACC_ROWS // NS
DUMMY = 25024
NPAD = 50176
BLK = 1024
GRID = NPAD // BLK

_sc_mesh = plsc.VectorSubcoreMesh(core_axis_name="c", subcore_axis_name="s")
_sc_params = pltpu.CompilerParams(use_tc_tiling_on_sc=False)


@functools.partial(
    pl.kernel,
    out_type=jax.ShapeDtypeStruct((NC, ACC_ROWS, LANES), jnp.float32),
    mesh=_sc_mesh,
    scratch_types=[
        pltpu.VMEM((CHUNK,), jnp.int32),
        pltpu.VMEM((CHUNK,), jnp.int32),
        pltpu.VMEM((CHUNK, LANES), jnp.float32),
        pltpu.VMEM((ROWS_PT, LANES), jnp.float32),
        pltpu.VMEM_SHARED((ACC_ROWS, LANES), jnp.float32),
    ],
    compiler_params=_sc_params,
)
def _deg_kernel(dst_hbm, cnt_hbm, dst_v, ldst_v, ones_v, zbuf_v, acc_s):
    c = lax.axis_index("c")
    s = lax.axis_index("s")
    base = c * HALF

    def fill_ones(i, carry):
        ones_v[i, :] = jnp.ones((LANES,), jnp.float32)
        return carry

    lax.fori_loop(0, CHUNK, fill_ones, 0)

    def fill_zero(i, carry):
        zbuf_v[i, :] = jnp.zeros((LANES,), jnp.float32)
        return carry

    lax.fori_loop(0, ROWS_PT, fill_zero, 0)
    pltpu.sync_copy(zbuf_v, acc_s.at[pl.ds(s * ROWS_PT, ROWS_PT)])
    plsc.subcore_barrier()

    def body(j, carry):
        off = (s * CPT + j) * CHUNK
        pltpu.sync_copy(dst_hbm.at[pl.ds(off, CHUNK)], dst_v)

        def remap(k, inner):
            d = dst_v[pl.ds(k * LANES, LANES)]
            ok = (d >= base) & (d < base + HALF)
            ldst_v[pl.ds(k * LANES, LANES)] = jnp.where(ok, d - base, DUMMY)
            return inner

        lax.fori_loop(0, CHUNK // LANES, remap, 0, unroll=True)
        pltpu.sync_copy(ones_v, acc_s.at[ldst_v], add=True)
        return carry

    lax.fori_loop(0, CPT, body, 0)
    plsc.subcore_barrier()
    pltpu.sync_copy(acc_s.at[pl.ds(s * ROWS_PT, ROWS_PT)],
                    cnt_hbm.at[c, pl.ds(s * ROWS_PT, ROWS_PT)])


@functools.partial(
    pl.kernel,
    out_type=jax.ShapeDtypeStruct((NC, ACC_ROWS, DIM), jnp.float32),
    mesh=_sc_mesh,
    scratch_types=[
        pltpu.VMEM((CHUNK,), jnp.int32),
        pltpu.VMEM((CHUNK,), jnp.int32),
        pltpu.VMEM((CHUNK,), jnp.int32),
        pltpu.VMEM((CHUNK, DIM), jnp.float32),
        pltpu.SemaphoreType.DMA,
        pltpu.VMEM_SHARED((ACC_ROWS, DIM), jnp.float32),
    ],
    compiler_params=_sc_params,
)
def _scatter_kernel(y_hbm, src_hbm, dst_hbm, out_hbm,
                    src_v, dst_v, ldst_v, rows_v, sem, acc_s):
    c = lax.axis_index("c")
    s = lax.axis_index("s")
    base = c * HALF
    r0 = s * ROWS_PT
    pltpu.sync_copy(y_hbm.at[pl.ds(base + r0, ROWS_PT)], acc_s.at[pl.ds(r0, ROWS_PT)])
    plsc.subcore_barrier()

    def body(j, carry):
        off = (s * CPT + j) * CHUNK
        pltpu.sync_copy(src_hbm.at[pl.ds(off, CHUNK)], src_v)
        pltpu.sync_copy(dst_hbm.at[pl.ds(off, CHUNK)], dst_v)
        pltpu.async_copy(y_hbm.at[src_v], rows_v, sem).wait()

        def remap(k, inner):
            d = dst_v[pl.ds(k * LANES, LANES)]
            ok = (d >= base) & (d < base + HALF)
            ldst_v[pl.ds(k * LANES, LANES)] = jnp.where(ok, d - base, DUMMY)
            return inner

        lax.fori_loop(0, CHUNK // LANES, remap, 0, unroll=True)
        pltpu.sync_copy(rows_v, acc_s.at[ldst_v], add=True)
        return carry

    lax.fori_loop(0, CPT, body, 0)
    plsc.subcore_barrier()
    pltpu.sync_copy(acc_s.at[pl.ds(r0, ROWS_PT)], out_hbm.at[c, pl.ds(r0, ROWS_PT)])


def _tcA_body(x_ref, cnt_ref, w_ref, y_ref, dis_ref):
    dis = lax.rsqrt(cnt_ref[...] + 1.0)
    xw = jnp.dot(x_ref[...], w_ref[...], preferred_element_type=jnp.float32)
    y_ref[...] = xw * dis
    dis_ref[...] = dis


def _tcB_body(acc_ref, dis_ref, b_ref, w_ref, out_ref, y_ref):
    dis = dis_ref[...]
    out = acc_ref[...] * dis + b_ref[0:1, :]
    out_ref[...] = out
    h = jnp.maximum(out, 0.0)
    y_ref[...] = jnp.dot(h, w_ref[...], preferred_element_type=jnp.float32) * dis


def _tcC_body(acc_ref, dis_ref, b_ref, out_ref):
    out_ref[...] = acc_ref[...] * dis_ref[...] + b_ref[0:1, :]


def _tcA(x_pad, cnt_pad, W1):
    return pl.pallas_call(
        _tcA_body,
        grid=(GRID,),
        in_specs=[pl.BlockSpec((BLK, DIM), lambda i: (i, 0)),
                  pl.BlockSpec((BLK, 1), lambda i: (i, 0)),
                  pl.BlockSpec((DIM, DIM), lambda i: (0, 0))],
        out_specs=[pl.BlockSpec((BLK, DIM), lambda i: (i, 0)),
                   pl.BlockSpec((BLK, 1), lambda i: (i, 0))],
        out_shape=[jax.ShapeDtypeStruct((NPAD, DIM), jnp.float32),
                   jax.ShapeDtypeStruct((NPAD, 1), jnp.float32)],
    )(x_pad, cnt_pad, W1)


def _tcB(acc_cat, dis, b1b, W2):
    return pl.pallas_call(
        _tcB_body,
        grid=(GRID,),
        in_specs=[pl.BlockSpec((BLK, DIM), lambda i: (i, 0)),
                  pl.BlockSpec((BLK, 1), lambda i: (i, 0)),
                  pl.BlockSpec((8, DIM), lambda i: (0, 0)),
                  pl.BlockSpec((DIM, DIM), lambda i: (0, 0))],
        out_specs=[pl.BlockSpec((BLK, DIM), lambda i: (i, 0)),
                   pl.BlockSpec((BLK, DIM), lambda i: (i, 0))],
        out_shape=[jax.ShapeDtypeStruct((NPAD, DIM), jnp.float32),
                   jax.ShapeDtypeStruct((NPAD, DIM), jnp.float32)],
    )(acc_cat, dis, b1b, W2)


def _tcC(acc_cat, dis, b2b):
    return pl.pallas_call(
        _tcC_body,
        grid=(GRID,),
        in_specs=[pl.BlockSpec((BLK, DIM), lambda i: (i, 0)),
                  pl.BlockSpec((BLK, 1), lambda i: (i, 0)),
                  pl.BlockSpec((8, DIM), lambda i: (0, 0))],
        out_specs=pl.BlockSpec((BLK, DIM), lambda i: (i, 0)),
        out_shape=jax.ShapeDtypeStruct((NPAD, DIM), jnp.float32),
    )(acc_cat, dis, b2b)


def _cat_acc(acc):
    return jnp.concatenate(
        [acc[0, :HALF], acc[1, :HALF],
         jnp.zeros((NPAD - N_NODES, DIM), jnp.float32)], axis=0)


def kernel(edge_index, edge_weight, emb_users, emb_items, W1, b1, W2, b2):
    del edge_weight
    src = edge_index[0].astype(jnp.int32)
    dst = edge_index[1].astype(jnp.int32)
    pad_e = E_PAD - E_EDGES
    src_p = jnp.concatenate([src, jnp.full((pad_e,), N_NODES, jnp.int32)])
    dst_p = jnp.concatenate([dst, jnp.full((pad_e,), 2 ** 28, jnp.int32)])

    x = jnp.concatenate([emb_users, emb_items], axis=0)
    x_pad = jnp.concatenate([x, jnp.zeros((NPAD - N_NODES, DIM), jnp.float32)])

    cnt = _deg_kernel(dst_p)
    cnt_col = jnp.concatenate([cnt[0, :HALF, 0], cnt[1, :HALF, 0],
                               jnp.zeros((NPAD - N_NODES,), jnp.float32)])
    cnt_pad = cnt_col.reshape(NPAD, 1)

    y1, dis = _tcA(x_pad, cnt_pad, W1)
    acc1 = _scatter_kernel(y1, src_p, dst_p)
    b1b = jnp.broadcast_to(b1.reshape(1, DIM), (8, DIM))
    out1, y2 = _tcB(_cat_acc(acc1), dis, b1b, W2)
    acc2 = _scatter_kernel(y2, src_p, dst_p)
    b2b = jnp.broadcast_to(b2.reshape(1, DIM), (8, DIM))
    out2 = _tcC(_cat_acc(acc2), dis, b2b)

    return (x, out1[:N_NODES], out2[:N_NODES])
